# baseline (speedup 1.0000x reference)
import functools

import jax
import jax.numpy as jnp
from jax import lax
from jax.experimental import pallas as pl
from jax.experimental.pallas import tpu as pltpu
from jax.experimental.pallas import tpu_sc as plsc

_NC = 2
_NS = 16
_LANES = 16
_NW = _NC * _NS
_NBUF = 3
_C = 200


def _repack(table):
    V, D = table.shape
    BLK = 2048

    def body(t_ref, o_ref):
        blk = t_ref[...]
        o_ref[...] = jnp.concatenate([blk, blk], axis=1)

    return pl.pallas_call(
        body,
        grid=(V // BLK,),
        in_specs=[pl.BlockSpec((BLK, D), lambda i: (i, 0))],
        out_specs=pl.BlockSpec((BLK, 2 * D), lambda i: (i, 0)),
        out_shape=jax.ShapeDtypeStruct((V, 2 * D), jnp.float32),
    )(table)

@functools.partial(jax.jit, static_argnums=())
def kernel(x, table, pos_enc):
    B, T = x.shape
    V, D = table.shape
    W = 2 * D
    scale = float(D) ** 0.5

    seq_per_w = B // _NW
    n_chunks = seq_per_w * (T // _C)
    half = T // _C

    tp = _repack(table)

    mesh = plsc.VectorSubcoreMesh(core_axis_name="c", subcore_axis_name="s")

    @functools.partial(
        pl.kernel,
        out_type=jax.ShapeDtypeStruct((B, T, W), jnp.float32),
        mesh=mesh,
        scratch_types=[
            pltpu.VMEM((seq_per_w, T), jnp.int32),
            pltpu.VMEM((T, D), jnp.float32),
            pltpu.VMEM((_NBUF, _C, W), jnp.float32),
            pltpu.SemaphoreType.DMA((_NBUF,)),
            pltpu.SemaphoreType.DMA((_NBUF,)),
        ],
        compiler_params=pltpu.CompilerParams(use_tc_tiling_on_sc=False),
    )
    def sc_embed(x_hbm, tp_hbm, pos_hbm, out_hbm, idx_v, pos_v, rows_v, gsem, osem):
        wid = lax.axis_index("s") * _NC + lax.axis_index("c")
        w_base = wid * seq_per_w
        pltpu.sync_copy(x_hbm.at[pl.ds(w_base, seq_per_w)], idx_v)
        pltpu.sync_copy(pos_hbm, pos_v)

        def out_slices(g):
            return out_hbm.at[w_base + g, :, pl.ds(0, D)]

        def idx_slice(g):
            return idx_v.at[g]

        def start_gather(g, b):
            pltpu.async_copy(tp_hbm.at[idx_slice(g)], rows_v.at[b], gsem.at[b])

        def wait_gather(g, b):
            pltpu.make_async_copy(
                tp_hbm.at[idx_slice(g)], rows_v.at[b], gsem.at[b]
            ).wait()

        def start_out(g, b):
            pltpu.async_copy(rows_v.at[b, :, pl.ds(0, D)], out_slices(g), osem.at[b])

        def wait_out(g, b):
            pltpu.make_async_copy(
                rows_v.at[b, :, pl.ds(0, D)], out_slices(g), osem.at[b]
            ).wait()

        start_gather(0, 0)
        start_gather(1, 1)

        def chunk_body(g, carry):
            b = lax.rem(g, _NBUF)
            wait_gather(g, b)

            @plsc.parallel_loop(0, _C, unroll=4)
            def _compute(r):
                for c in range(D // _LANES):
                    sl = pl.ds(c * _LANES, _LANES)
                    rows_v[b, r, sl] = rows_v[b, r, sl] * scale + pos_v[r, sl]

            start_out(g, b)

            b2 = lax.rem(g + 2, _NBUF)

            @pl.when(g >= 1)
            def _():
                wait_out(g - 1, b2)

            @pl.when(g + 2 < n_chunks)
            def _():
                start_gather(g + 2, b2)

            return carry

        lax.fori_loop(0, n_chunks, chunk_body, 0)
        wait_out(n_chunks - 1, lax.rem(n_chunks - 1, _NBUF))

    out = sc_embed(x, tp, pos_enc)
    return out[:, :, :D]


# speedup vs baseline: 1.4123x; 1.4123x over previous
import functools

import jax
import jax.numpy as jnp
from jax import lax
from jax.experimental import pallas as pl
from jax.experimental.pallas import tpu as pltpu
from jax.experimental.pallas import tpu_sc as plsc

_NC = 2
_NS = 16
_LANES = 16
_NW = _NC * _NS
_NBUF = 4


@functools.partial(jax.jit, static_argnums=())
def kernel(x, table, pos_enc):
    B, T = x.shape
    V, D = table.shape
    W = 2 * D
    scale = float(D) ** 0.5

    seq_per_w = B // _NW
    n_chunks = seq_per_w

    mesh = plsc.VectorSubcoreMesh(core_axis_name="c", subcore_axis_name="s")

    @functools.partial(
        pl.kernel,
        out_type=jax.ShapeDtypeStruct((B, T, W), jnp.float32),
        mesh=mesh,
        scratch_types=[
            pltpu.VMEM((seq_per_w, T), jnp.int32),
            pltpu.VMEM((T, D), jnp.float32),
            pltpu.VMEM((_NBUF, T, D), jnp.float32),
            pltpu.SemaphoreType.DMA((_NBUF,)),
            pltpu.SemaphoreType.DMA((_NBUF,)),
        ],
        compiler_params=pltpu.CompilerParams(use_tc_tiling_on_sc=False),
    )
    def sc_embed(x_hbm, table_hbm, pos_hbm, out_hbm, idx_v, pos_v, rows_v, gsem, osem):
        wid = lax.axis_index("s") * _NC + lax.axis_index("c")
        w_base = wid * seq_per_w
        pltpu.sync_copy(x_hbm.at[pl.ds(w_base, seq_per_w)], idx_v)
        pltpu.sync_copy(pos_hbm, pos_v)

        def start_gather(g, b):
            pltpu.async_copy(table_hbm.at[idx_v.at[g]], rows_v.at[b], gsem.at[b])

        def wait_gather(g, b):
            pltpu.make_async_copy(
                table_hbm.at[idx_v.at[g]], rows_v.at[b], gsem.at[b]
            ).wait()

        def start_out(g, b):
            pltpu.async_copy(
                rows_v.at[b], out_hbm.at[w_base + g, :, pl.ds(0, D)], osem.at[b]
            )

        def wait_out(g, b):
            pltpu.make_async_copy(
                rows_v.at[b], out_hbm.at[w_base + g, :, pl.ds(0, D)], osem.at[b]
            ).wait()

        start_gather(0, 0)
        start_gather(1, 1)

        def chunk_body(g, carry):
            b = lax.rem(g, _NBUF)
            wait_gather(g, b)

            @plsc.parallel_loop(0, T, unroll=4)
            def _compute(r):
                for c in range(D // _LANES):
                    sl = pl.ds(c * _LANES, _LANES)
                    rows_v[b, r, sl] = rows_v[b, r, sl] * scale + pos_v[r, sl]

            start_out(g, b)

            b2 = lax.rem(g + 2, _NBUF)

            @pl.when(g >= 2)
            def _():
                wait_out(g - 2, b2)

            @pl.when(g + 2 < n_chunks)
            def _():
                start_gather(g + 2, b2)

            return carry

        lax.fori_loop(0, n_chunks, chunk_body, 0)
        wait_out(n_chunks - 2, lax.rem(n_chunks - 2, _NBUF))
        wait_out(n_chunks - 1, lax.rem(n_chunks - 1, _NBUF))

    out = sc_embed(x, table, pos_enc)
    return out[:, :, :D]
